# baseline (speedup 1.0000x reference)
import functools

import jax
import jax.numpy as jnp
from jax import lax
from jax.experimental import pallas as pl
from jax.experimental.pallas import tpu as pltpu
from jax.experimental.pallas import tpu_sc as plsc

Q = 1024
D_IN = 768
D_PROJ = 384
K_ENTRIES = 100000
TOP_K = 8
K_BLK = 2048
NB = (K_ENTRIES + K_BLK - 1) // K_BLK
K_PAD = NB * K_BLK
SB = 256
SPB = K_BLK // SB
NSB = K_PAD // SB
QC = 256

_INF = float("inf")
_IMAX = 2**31 - 1


def _dist_body(img_ref, w_ref, b_ref, keys_ref, d_ref, sb_ref,
               proj_ref, qsq_ref):
    k = pl.program_id(0)

    @pl.when(k == 0)
    def _init():
        p = lax.dot_general(
            img_ref[...], w_ref[...],
            dimension_numbers=(((1,), (1,)), ((), ())),
            preferred_element_type=jnp.float32,
        ) + b_ref[...]
        proj_ref[...] = p
        qsq_ref[...] = jnp.sum(p * p, axis=1, keepdims=True)

    keys = keys_ref[...]
    ksq = jnp.sum(keys * keys, axis=1, keepdims=True)
    ksq_row = jnp.transpose(ksq)
    m = lax.dot_general(
        proj_ref[...], keys,
        dimension_numbers=(((1,), (1,)), ((), ())),
        preferred_element_type=jnp.float32,
    )
    d = (qsq_ref[...] - 2.0 * m) + ksq_row

    col = lax.broadcasted_iota(jnp.int32, (Q, K_BLK), 1)
    d = jnp.where(col + k * K_BLK >= K_ENTRIES, _INF, d)

    d_ref[...] = d.reshape(Q, SPB, SB)
    mins = jnp.concatenate(
        [jnp.min(d[:, s * SB:(s + 1) * SB], axis=1, keepdims=True)
         for s in range(SPB)], axis=1)
    sb_ref[...] = mins[None, :, :]


def _dists_and_sbmins(image_emb, W, b2, keys_pad):
    return pl.pallas_call(
        _dist_body,
        grid=(NB,),
        in_specs=[
            pl.BlockSpec((Q, D_IN), lambda k: (0, 0)),
            pl.BlockSpec((D_PROJ, D_IN), lambda k: (0, 0)),
            pl.BlockSpec((1, D_PROJ), lambda k: (0, 0)),
            pl.BlockSpec((K_BLK, D_PROJ), lambda k: (k, 0)),
        ],
        out_specs=[
            pl.BlockSpec((Q, SPB, SB), lambda k: (0, k, 0)),
            pl.BlockSpec((1, Q, SPB), lambda k: (k, 0, 0)),
        ],
        out_shape=[
            jax.ShapeDtypeStruct((Q, NSB, SB), jnp.float32),
            jax.ShapeDtypeStruct((NB, Q, SPB), jnp.float32),
        ],
        scratch_shapes=[
            pltpu.VMEM((Q, D_PROJ), jnp.float32),
            pltpu.VMEM((Q, 1), jnp.float32),
        ],
        compiler_params=pltpu.CompilerParams(
            dimension_semantics=("arbitrary",),
        ),
    )(image_emb, W, b2, keys_pad)


def _sbtopk_body(sbmin_ref, sb_out_ref, row_out_ref):
    work = sbmin_ref[...]
    col = lax.broadcasted_iota(jnp.int32, (Q, NSB), 1)
    q_iota = lax.broadcasted_iota(jnp.int32, (Q, 1), 0)
    sbs, rows = [], []
    for _ in range(TOP_K):
        mn = jnp.min(work, axis=1, keepdims=True)
        pos = jnp.min(jnp.where(work == mn, col, _IMAX), axis=1, keepdims=True)
        sbs.append(pos)
        rows.append(q_iota * NSB + pos)
        work = jnp.where(col == pos, _INF, work)
    sb_out_ref[...] = jnp.concatenate(sbs, axis=1)
    row_out_ref[...] = jnp.concatenate(rows, axis=1)


def _sb_topk(sbmin):
    return pl.pallas_call(
        _sbtopk_body,
        out_shape=[
            jax.ShapeDtypeStruct((Q, TOP_K), jnp.int32),
            jax.ShapeDtypeStruct((Q, TOP_K), jnp.int32),
        ],
    )(sbmin)


def _final_body(cand_ref, sb_ref, out_ref):
    work = cand_ref[...]
    sb = sb_ref[...]
    lane = lax.broadcasted_iota(jnp.int32, (QC, SB), 1)
    gidx = jnp.concatenate(
        [sb[:, c:c + 1] * SB + lane for c in range(TOP_K)], axis=1)
    outs = []
    for _ in range(TOP_K):
        mn = jnp.min(work, axis=1, keepdims=True)
        imn = jnp.min(jnp.where(work == mn, gidx, _IMAX), axis=1, keepdims=True)
        outs.append(imn)
        work = jnp.where((work == mn) & (gidx == imn), _INF, work)
    out_ref[...] = jnp.concatenate(outs, axis=1)


def _final_topk(cand, sb_ids):
    return pl.pallas_call(
        _final_body,
        grid=(Q // QC,),
        in_specs=[
            pl.BlockSpec((QC, TOP_K * SB), lambda i: (i, 0)),
            pl.BlockSpec((QC, TOP_K), lambda i: (i, 0)),
        ],
        out_specs=pl.BlockSpec((QC, TOP_K), lambda i: (i, 0)),
        out_shape=jax.ShapeDtypeStruct((Q, TOP_K), jnp.int32),
    )(cand, sb_ids)


_NC, _NS = 2, 16
_NW = _NC * _NS
_CH = 128


def _gather_rows_sc(table, flat_idx):
    n_rows, d = table.shape
    b = flat_idx.shape[0]
    bpw = b // _NW
    mesh = plsc.VectorSubcoreMesh(core_axis_name="c", subcore_axis_name="s")

    @functools.partial(
        pl.kernel, mesh=mesh,
        out_type=jax.ShapeDtypeStruct((b, d), jnp.float32),
        scratch_types=[
            pltpu.VMEM((_CH,), jnp.int32),
            pltpu.VMEM((_CH, d), jnp.float32),
            pltpu.SemaphoreType.DMA,
        ],
    )
    def gather(table_hbm, idx_hbm, out_hbm, idx_v, rows_v, sem):
        wid = lax.axis_index("s") * _NC + lax.axis_index("c")
        for r in range(bpw // _CH):
            base = wid * bpw + r * _CH
            pltpu.sync_copy(idx_hbm.at[pl.ds(base, _CH)], idx_v)
            pltpu.async_copy(table_hbm.at[idx_v], rows_v, sem).wait()
            pltpu.sync_copy(rows_v, out_hbm.at[pl.ds(base, _CH)])

    return gather(table, flat_idx)


def kernel(image_emb, W, b, index_keys, entry_embs):
    b2 = b.reshape(1, D_PROJ)
    d_full, sbmin3 = _dists_and_sbmins(image_emb, W, b2, index_keys)
    sbmin = jnp.transpose(sbmin3, (1, 0, 2)).reshape(Q, NSB)
    sb_ids, row_ids = _sb_topk(sbmin)
    cand = _gather_rows_sc(d_full.reshape(Q * NSB, SB), row_ids.reshape(-1))
    idx = _final_topk(cand.reshape(Q, TOP_K * SB), sb_ids)
    rows = _gather_rows_sc(entry_embs, idx.reshape(-1))
    return rows.reshape(Q, TOP_K, D_PROJ)
